# baseline (speedup 1.0000x reference)
import functools

import jax
import jax.numpy as jnp
from jax import lax
from jax.experimental import pallas as pl
from jax.experimental.pallas import tpu as pltpu
from jax.experimental.pallas import tpu_sc as plsc

_BATCH = 4096
_FIELDS = 26
_DIM = 16
_NC = 2
_NS = 16
_NW = _NC * _NS
_ROWS_PER_W = _BATCH // _NW
_IDX_PER_W = _ROWS_PER_W * _FIELDS
_GROUPS = _ROWS_PER_W // 16


def _lr_body(idx_hbm, table_hbm, out_hbm, idx_v, vals_v, sums_v, sem):
    wid = lax.axis_index("s") * _NC + lax.axis_index("c")
    icps = [
        pltpu.async_copy(
            idx_hbm.at[pl.ds(j * _BATCH + wid * _ROWS_PER_W, _ROWS_PER_W)],
            idx_v.at[pl.ds(j * 128, 128)],
            sem,
        )
        for j in range(_FIELDS)
    ]
    for c in icps:
        c.wait()
    copies = [
        pltpu.async_copy(
            table_hbm.at[idx_v.at[pl.ds(j * 128, 128)]],
            vals_v.at[pl.ds(j * 128, 128)],
            sem,
        )
        for j in range(_FIELDS)
    ]
    for c in copies:
        c.wait()
    for g in range(_GROUPS):
        acc = vals_v[pl.ds(g * 16, 16)]
        for f in range(1, _FIELDS):
            acc = acc + vals_v[pl.ds(f * 128 + g * 16, 16)]
        sums_v[pl.ds(g * 16, 16)] = acc
    pltpu.sync_copy(sums_v, out_hbm.at[pl.ds(wid * _ROWS_PER_W, _ROWS_PER_W)])


_lr_call = functools.partial(
    pl.kernel,
    mesh=plsc.VectorSubcoreMesh(core_axis_name="c", subcore_axis_name="s"),
    out_type=jax.ShapeDtypeStruct((_BATCH,), jnp.float32),
    scratch_types=[
        pltpu.VMEM((_IDX_PER_W,), jnp.int32),
        pltpu.VMEM((_IDX_PER_W,), jnp.float32),
        pltpu.VMEM((_ROWS_PER_W,), jnp.float32),
        pltpu.SemaphoreType.DMA,
    ],
)(_lr_body)


_FB = _FIELDS * _DIM
_BB = 2048


def _fm_body(x_ref, o_ref):
    x = x_ref[...]
    s = x[0:_DIM, :]
    for f in range(1, _FIELDS):
        s = s + x[f * _DIM:(f + 1) * _DIM, :]
    sq_of_sum = jnp.sum(s * s, axis=0, keepdims=True)
    sum_of_sq = jnp.sum(x * x, axis=0, keepdims=True)
    o_ref[...] = 0.5 * (sq_of_sum - sum_of_sq)


_fm_call = pl.pallas_call(
    _fm_body,
    grid=(_BATCH // _BB,),
    in_specs=[pl.BlockSpec((_FB, _BB), lambda i: (0, i))],
    out_specs=pl.BlockSpec((1, _BB), lambda i: (0, i)),
    out_shape=jax.ShapeDtypeStruct((1, _BATCH), jnp.float32),
)


def kernel(X, feature_emb, lr_table, bias):
    idx = X.astype(jnp.int32).T.reshape(-1)
    table = lr_table.reshape(-1)
    lr_sums = _lr_call(idx, table)
    fm = _fm_call(feature_emb.transpose(1, 2, 0).reshape(_FB, _BATCH))
    return fm[0][:, None] + lr_sums[:, None] + bias


# speedup vs baseline: 1.0361x; 1.0361x over previous
import functools

import jax
import jax.numpy as jnp
from jax import lax
from jax.experimental import pallas as pl
from jax.experimental.pallas import tpu as pltpu
from jax.experimental.pallas import tpu_sc as plsc

_BATCH = 4096
_FIELDS = 26
_DIM = 16
_NC = 2
_NS = 16
_NW = _NC * _NS
_ROWS_PER_W = _BATCH // _NW
_IDX_PER_W = _ROWS_PER_W * _FIELDS
_GROUPS = _ROWS_PER_W // 16


def _lr_body(idx_hbm, table_hbm, out_hbm, idx_v, vals_v, sums_v, sem):
    wid = lax.axis_index("s") * _NC + lax.axis_index("c")
    pltpu.sync_copy(idx_hbm.at[:, pl.ds(wid * _ROWS_PER_W, _ROWS_PER_W)], idx_v)
    copies = [
        pltpu.async_copy(
            table_hbm.at[idx_v.at[j]],
            vals_v.at[pl.ds(j * 128, 128)],
            sem,
        )
        for j in range(_FIELDS)
    ]
    for c in copies:
        c.wait()
    for g in range(_GROUPS):
        acc = vals_v[pl.ds(g * 16, 16)]
        for f in range(1, _FIELDS):
            acc = acc + vals_v[pl.ds(f * 128 + g * 16, 16)]
        sums_v[pl.ds(g * 16, 16)] = acc
    pltpu.sync_copy(sums_v, out_hbm.at[pl.ds(wid * _ROWS_PER_W, _ROWS_PER_W)])


_lr_call = functools.partial(
    pl.kernel,
    mesh=plsc.VectorSubcoreMesh(core_axis_name="c", subcore_axis_name="s"),
    out_type=jax.ShapeDtypeStruct((_BATCH,), jnp.float32),
    scratch_types=[
        pltpu.VMEM((_FIELDS, _ROWS_PER_W), jnp.int32),
        pltpu.VMEM((_IDX_PER_W,), jnp.float32),
        pltpu.VMEM((_ROWS_PER_W,), jnp.float32),
        pltpu.SemaphoreType.DMA,
    ],
)(_lr_body)


_FB = _FIELDS * _DIM
_BB = 1024


def _fm_body(x_ref, o_ref):
    x = x_ref[...]
    s = x[0:_DIM, :]
    for f in range(1, _FIELDS):
        s = s + x[f * _DIM:(f + 1) * _DIM, :]
    sq_of_sum = jnp.sum(s * s, axis=0, keepdims=True)
    sum_of_sq = jnp.sum(x * x, axis=0, keepdims=True)
    o_ref[...] = 0.5 * (sq_of_sum - sum_of_sq)


_fm_call = pl.pallas_call(
    _fm_body,
    grid=(_BATCH // _BB,),
    in_specs=[pl.BlockSpec((_FB, _BB), lambda i: (0, i))],
    out_specs=pl.BlockSpec((1, _BB), lambda i: (0, i)),
    out_shape=jax.ShapeDtypeStruct((1, _BATCH), jnp.float32),
)


def kernel(X, feature_emb, lr_table, bias):
    idx = X.astype(jnp.int32).T
    table = lr_table.reshape(-1)
    lr_sums = _lr_call(idx, table)
    fm = _fm_call(feature_emb.transpose(1, 2, 0).reshape(_FB, _BATCH))
    return fm[0][:, None] + lr_sums[:, None] + bias


# speedup vs baseline: 1.0369x; 1.0008x over previous
import functools

import jax
import jax.numpy as jnp
from jax import lax
from jax.experimental import pallas as pl
from jax.experimental.pallas import tpu as pltpu
from jax.experimental.pallas import tpu_sc as plsc

_BATCH = 4096
_FIELDS = 26
_DIM = 16
_NC = 2
_NS = 16
_NW = _NC * _NS
_ROWS_PER_W = _BATCH // _NW
_IDX_PER_W = _ROWS_PER_W * _FIELDS
_GROUPS = _ROWS_PER_W // 16


def _lr_body(idx_hbm, table_hbm, out_hbm, idx_v, vals_v, sums_v, sem):
    wid = lax.axis_index("s") * _NC + lax.axis_index("c")
    pltpu.sync_copy(idx_hbm.at[:, pl.ds(wid * _ROWS_PER_W, _ROWS_PER_W)], idx_v)
    copies = [
        pltpu.async_copy(
            table_hbm.at[idx_v.at[j]],
            vals_v.at[pl.ds(j * 128, 128)],
            sem,
        )
        for j in range(_FIELDS)
    ]
    for c in copies:
        c.wait()
    for g in range(_GROUPS):
        acc = vals_v[pl.ds(g * 16, 16)]
        for f in range(1, _FIELDS):
            acc = acc + vals_v[pl.ds(f * 128 + g * 16, 16)]
        sums_v[pl.ds(g * 16, 16)] = acc
    pltpu.sync_copy(sums_v, out_hbm.at[pl.ds(wid * _ROWS_PER_W, _ROWS_PER_W)])


_lr_call = functools.partial(
    pl.kernel,
    mesh=plsc.VectorSubcoreMesh(core_axis_name="c", subcore_axis_name="s"),
    out_type=jax.ShapeDtypeStruct((_BATCH,), jnp.float32),
    scratch_types=[
        pltpu.VMEM((_FIELDS, _ROWS_PER_W), jnp.int32),
        pltpu.VMEM((_IDX_PER_W,), jnp.float32),
        pltpu.VMEM((_ROWS_PER_W,), jnp.float32),
        pltpu.SemaphoreType.DMA,
    ],
)(_lr_body)


_FB = _FIELDS * _DIM
_BB = 1024


def _fm_body(x_ref, o_ref):
    x = x_ref[...]
    s = x[0:_DIM, :]
    for f in range(1, _FIELDS):
        s = s + x[f * _DIM:(f + 1) * _DIM, :]
    sq_of_sum = jnp.sum(s * s, axis=0, keepdims=True)
    sum_of_sq = jnp.sum(x * x, axis=0, keepdims=True)
    o_ref[...] = 0.5 * (sq_of_sum - sum_of_sq)


_fm_call = pl.pallas_call(
    _fm_body,
    grid=(_BATCH // _BB,),
    in_specs=[pl.BlockSpec((_FB, _BB), lambda i: (0, i))],
    out_specs=pl.BlockSpec((1, _BB), lambda i: (0, i)),
    out_shape=jax.ShapeDtypeStruct((1, _BATCH), jnp.float32),
)


def kernel(X, feature_emb, lr_table, bias):
    idx = X.astype(jnp.int32).T
    table = lr_table.reshape(-1)
    lr_sums = _lr_call(idx, table)
    fm = _fm_call(feature_emb.transpose(1, 2, 0).reshape(_FB, _BATCH))
    return (fm[0] + lr_sums + bias[0])[:, None]
